# baseline (speedup 1.0000x reference)
import functools

import jax
import jax.numpy as jnp
from jax import lax
from jax.experimental import pallas as pl
from jax.experimental.pallas import tpu as pltpu
from jax.experimental.pallas import tpu_sc as plsc

N_NODES = 10000
D_FEAT = 256
N_GRAPHS = 128
D_GLOBAL = 64
HIDDEN = 256
OUT = 256

NC = 2
NS = 16
NW = NC * NS

ROWS = 312
TAIL = N_NODES - NW * ROWS
CW = 16
LPR = D_FEAT // 16
CHUNK = 112
NCHUNK = 3
RPAD = NCHUNK * CHUNK
TRASH = N_GRAPHS

_sc_mesh = plsc.VectorSubcoreMesh(core_axis_name="c", subcore_axis_name="s")


@functools.partial(
    pl.kernel,
    out_type=[
        jax.ShapeDtypeStruct((NW, N_GRAPHS, D_FEAT), jnp.float32),
        jax.ShapeDtypeStruct((NW, N_GRAPHS, CW), jnp.float32),
    ],
    mesh=_sc_mesh,
    scratch_types=[
        pltpu.VMEM((RPAD,), jnp.int32),
        pltpu.VMEM((CHUNK, D_FEAT), jnp.float32),
        pltpu.VMEM((N_GRAPHS + 8, D_FEAT), jnp.float32),
        pltpu.VMEM((N_GRAPHS + 8, CW), jnp.float32),
    ],
)
def _sc_segment_sum(x_hbm, batch_hbm, parts_hbm, cnts_hbm,
                    idx_v, rows_v, acc_v, cnt_v):
    cid = lax.axis_index("c")
    sid = lax.axis_index("s")
    wid = sid * NC + cid
    off = wid * ROWS
    n = jnp.where(wid == NW - 1, ROWS + TAIL, ROWS)

    pltpu.sync_copy(batch_hbm.at[pl.ds(off, ROWS)], idx_v.at[pl.ds(0, ROWS)])

    @pl.when(wid == NW - 1)
    def _stage_tail_ids():
        pltpu.sync_copy(batch_hbm.at[pl.ds(NW * ROWS, TAIL)],
                        idx_v.at[pl.ds(ROWS, TAIL)])

    zv = jnp.zeros((16,), jnp.float32)

    def _zero(r, _):
        for c in range(LPR):
            acc_v[r, pl.ds(c * 16, 16)] = zv
        cnt_v[r, :] = zv
        return 0

    lax.fori_loop(0, N_GRAPHS + 8, _zero, 0, unroll=4)

    one_v = jnp.ones((16,), jnp.float32)
    lane = lax.iota(jnp.int32, 16)

    def _chunk(c_i, _):
        coff = c_i * CHUNK

        @pl.when(c_i < 2)
        def _stage_full():
            pltpu.sync_copy(x_hbm.at[pl.ds(off + coff, CHUNK)], rows_v)

        @pl.when(c_i == 2)
        def _stage_last():
            pltpu.sync_copy(x_hbm.at[pl.ds(off + 2 * CHUNK, ROWS - 2 * CHUNK)],
                            rows_v.at[pl.ds(0, ROWS - 2 * CHUNK)])

            @pl.when(wid == NW - 1)
            def _stage_tail_rows():
                pltpu.sync_copy(
                    x_hbm.at[pl.ds(NW * ROWS, TAIL)],
                    rows_v.at[pl.ds(ROWS - 2 * CHUNK, TAIL)])

        def _group(g, _):
            segv = idx_v[pl.ds(coff + g * 16, 16)]
            rowid = coff + g * 16 + lane
            segv = jnp.where(rowid < n, segv, TRASH)
            for j in range(16):
                seg = segv[j]
                r = g * 16 + j
                for c in range(LPR):
                    plsc.addupdate(acc_v.at[seg, pl.ds(c * 16, 16)],
                                   rows_v[r, pl.ds(c * 16, 16)])
                plsc.addupdate(cnt_v.at[seg, :], one_v)
            return 0

        lax.fori_loop(0, CHUNK // 16, _group, 0)
        return 0

    lax.fori_loop(0, NCHUNK, _chunk, 0)

    pltpu.sync_copy(acc_v.at[pl.ds(0, N_GRAPHS)], parts_hbm.at[wid])
    pltpu.sync_copy(cnt_v.at[pl.ds(0, N_GRAPHS)], cnts_hbm.at[wid])


def _tc_mlp_body(parts_ref, cnts_ref, u_ref, w1u_ref, w1m_ref, b1_ref,
                 w2_ref, b2_ref, w3_ref, b3_ref, out_ref):
    sums = jnp.sum(parts_ref[...], axis=0)
    counts = jnp.sum(cnts_ref[...], axis=(0, 2)) * (1.0 / CW)
    mean = sums * (1.0 / jnp.maximum(counts, 1.0))[:, None]
    h = u_ref[...] @ w1u_ref[...] + mean @ w1m_ref[...] + b1_ref[...]
    h = jnp.maximum(h, 0.0)
    h = jnp.maximum(h @ w2_ref[...] + b2_ref[...], 0.0)
    out_ref[...] = h @ w3_ref[...] + b3_ref[...]


def _tc_mlp(parts, cnts, u, W1u, W1m, b1, W2, b2, W3, b3):
    return pl.pallas_call(
        _tc_mlp_body,
        out_shape=jax.ShapeDtypeStruct((N_GRAPHS, OUT), jnp.float32),
    )(parts, cnts, u, W1u, W1m, b1, W2, b2, W3, b3)


@jax.jit
def _run(x, batch, u, W1u, W1m, b1, W2, b2, W3, b3):
    parts, cnts = _sc_segment_sum(x, batch)
    return _tc_mlp(parts, cnts, u, W1u, W1m, b1, W2, b2, W3, b3)


def kernel(x, edge_index, edge_attr, u, batch, W1, b1, W2, b2, W3, b3):
    del edge_index, edge_attr
    return _run(x, batch, u, W1[:D_GLOBAL], W1[D_GLOBAL:],
                b1.reshape(1, -1), W2, b2.reshape(1, -1), W3,
                b3.reshape(1, -1))


# speedup vs baseline: 1.2255x; 1.2255x over previous
import functools

import jax
import jax.numpy as jnp
from jax import lax
from jax.experimental import pallas as pl
from jax.experimental.pallas import tpu as pltpu
from jax.experimental.pallas import tpu_sc as plsc

N_NODES = 10000
D_FEAT = 256
N_GRAPHS = 128
D_GLOBAL = 64
HIDDEN = 256
OUT = 256

NC = 2
NS = 16
NW = NC * NS

ROWS = 312
TAIL = N_NODES - NW * ROWS
CW = 16
LPR = D_FEAT // 16
CHUNK = 112
NCHUNK = 3
RPAD = NCHUNK * CHUNK
TRASH = N_GRAPHS

_sc_mesh = plsc.VectorSubcoreMesh(core_axis_name="c", subcore_axis_name="s")


@functools.partial(
    pl.kernel,
    out_type=[
        jax.ShapeDtypeStruct((NW, N_GRAPHS, D_FEAT), jnp.float32),
        jax.ShapeDtypeStruct((NW, N_GRAPHS, CW), jnp.float32),
    ],
    mesh=_sc_mesh,
    scratch_types=[
        pltpu.VMEM((RPAD,), jnp.int32),
        pltpu.VMEM((CHUNK, D_FEAT), jnp.float32),
        pltpu.VMEM((N_GRAPHS + 8, D_FEAT), jnp.float32),
        pltpu.VMEM((N_GRAPHS + 8, CW), jnp.float32),
    ],
)
def _sc_segment_sum(x_hbm, batch_hbm, parts_hbm, cnts_hbm,
                    idx_v, rows_v, acc_v, cnt_v):
    cid = lax.axis_index("c")
    sid = lax.axis_index("s")
    wid = sid * NC + cid
    off = wid * ROWS
    n = jnp.where(wid == NW - 1, ROWS + TAIL, ROWS)

    pltpu.sync_copy(batch_hbm.at[pl.ds(off, ROWS)], idx_v.at[pl.ds(0, ROWS)])

    @pl.when(wid == NW - 1)
    def _stage_tail_ids():
        pltpu.sync_copy(batch_hbm.at[pl.ds(NW * ROWS, TAIL)],
                        idx_v.at[pl.ds(ROWS, TAIL)])

    zv = jnp.zeros((16,), jnp.float32)

    def _zero(r, _):
        cnt_v[r, :] = zv
        return 0

    lax.fori_loop(0, N_GRAPHS + 8, _zero, 0, unroll=8)

    lane = lax.iota(jnp.int32, 16)
    one16 = jnp.ones((16,), jnp.float32)

    def _row_step(seg, row, carry):
        cur, rl = carry[0], carry[1]
        accs = carry[2:]
        flush = seg != cur
        new_cur = jnp.where(flush, seg, cur)
        new_rl = jnp.where(flush, jnp.int32(1), rl + 1)
        keep = jnp.where(flush, 0.0, 1.0)
        new_accs = tuple(b + a * keep for a, b in zip(accs, row))
        for c in range(LPR):
            acc_v[new_cur, pl.ds(c * 16, 16)] = new_accs[c]
        cnt_v[new_cur, :] = one16 * new_rl.astype(jnp.float32)
        return (new_cur, new_rl) + new_accs

    def _chunk(c_i, carry):
        coff = c_i * CHUNK

        @pl.when(c_i < 2)
        def _stage_full():
            pltpu.sync_copy(x_hbm.at[pl.ds(off + coff, CHUNK)], rows_v)

        @pl.when(c_i == 2)
        def _stage_last():
            pltpu.sync_copy(x_hbm.at[pl.ds(off + 2 * CHUNK, ROWS - 2 * CHUNK)],
                            rows_v.at[pl.ds(0, ROWS - 2 * CHUNK)])

            @pl.when(wid == NW - 1)
            def _stage_tail_rows():
                pltpu.sync_copy(
                    x_hbm.at[pl.ds(NW * ROWS, TAIL)],
                    rows_v.at[pl.ds(ROWS - 2 * CHUNK, TAIL)])

        def _group(g, carry):
            segv = idx_v[pl.ds(coff + g * 16, 16)]
            rowid = coff + g * 16 + lane
            segv = jnp.where(rowid < n, segv, TRASH)
            for j in range(16):
                r = g * 16 + j
                row = [rows_v[r, pl.ds(c * 16, 16)] for c in range(LPR)]
                carry = _row_step(segv[j], row, carry)
            return carry

        return lax.fori_loop(0, CHUNK // 16, _group, carry)

    carry0 = (jnp.int32(TRASH), jnp.int32(0)) + tuple(
        jnp.zeros((16,), jnp.float32) for _ in range(LPR))
    lax.fori_loop(0, NCHUNK, _chunk, carry0)

    pltpu.sync_copy(acc_v.at[pl.ds(0, N_GRAPHS)], parts_hbm.at[wid])
    pltpu.sync_copy(cnt_v.at[pl.ds(0, N_GRAPHS)], cnts_hbm.at[wid])


def _tc_mlp_body(parts_ref, cnts_ref, u_ref, w1u_ref, w1m_ref, b1_ref,
                 w2_ref, b2_ref, w3_ref, b3_ref, out_ref):
    cnts = cnts_ref[...]
    mask_c = jnp.concatenate([cnts] * (D_FEAT // CW), axis=2)
    parts = jnp.where(mask_c > 0.0, parts_ref[...], 0.0)
    sums = jnp.sum(parts, axis=0)
    counts = jnp.sum(cnts, axis=(0, 2)) * (1.0 / CW)
    mean = sums * (1.0 / jnp.maximum(counts, 1.0))[:, None]
    h = u_ref[...] @ w1u_ref[...] + mean @ w1m_ref[...] + b1_ref[...]
    h = jnp.maximum(h, 0.0)
    h = jnp.maximum(h @ w2_ref[...] + b2_ref[...], 0.0)
    out_ref[...] = h @ w3_ref[...] + b3_ref[...]


def _tc_mlp(parts, cnts, u, W1u, W1m, b1, W2, b2, W3, b3):
    return pl.pallas_call(
        _tc_mlp_body,
        out_shape=jax.ShapeDtypeStruct((N_GRAPHS, OUT), jnp.float32),
    )(parts, cnts, u, W1u, W1m, b1, W2, b2, W3, b3)


@jax.jit
def _run(x, batch, u, W1u, W1m, b1, W2, b2, W3, b3):
    parts, cnts = _sc_segment_sum(x, batch)
    return _tc_mlp(parts, cnts, u, W1u, W1m, b1, W2, b2, W3, b3)


def kernel(x, edge_index, edge_attr, u, batch, W1, b1, W2, b2, W3, b3):
    del edge_index, edge_attr
    return _run(x, batch, u, W1[:D_GLOBAL], W1[D_GLOBAL:],
                b1.reshape(1, -1), W2, b2.reshape(1, -1), W3,
                b3.reshape(1, -1))
